# baseline (device time: 60033 ns/iter reference)
import jax
import jax.numpy as jnp
from jax import lax
from jax.experimental import pallas as pl
from jax.experimental.pallas import tpu as pltpu

N_DEV = 4
N_TOK = 2048
D_IN = 512
D_OUT = 1024
HALF = D_OUT // 2
Q = HALF // 2
N_EXP = 16
EXP_PER_DEV = N_EXP // N_DEV
CHUNK = N_TOK // N_DEV
N_HOPS = 2 * (N_DEV - 1)
N_STAGED = N_DEV


def kernel(x, router_W, route_idx, expert_W):
    def body(x_ref, rw_ref, idx_ref, ew_ref, out_ref,
             sb0, sb1, sb2, sb3, rb0, rb1, rb2, rb3,
             ss0, ss1, ss2, ss3, rs0, rs1, rs2, rs3):
        my = lax.axis_index("i")
        left = lax.rem(my + N_DEV - 1, N_DEV)
        right = lax.rem(my + 1, N_DEV)

        barrier_sem = pltpu.get_barrier_semaphore()
        for nbr in (left, right):
            pl.semaphore_signal(
                barrier_sem, inc=1,
                device_id=(nbr,), device_id_type=pl.DeviceIdType.MESH,
            )
        pl.semaphore_wait(barrier_sem, 2)

        streams = (
            (right, sb0, rb0, ss0, rs0, 0),
            (right, sb1, rb1, ss1, rs1, Q),
            (left, sb2, rb2, ss2, rs2, HALF),
            (left, sb3, rb3, ss3, rs3, HALF + Q),
        )
        cols = tuple(slice(off, off + Q) for off in (0, Q, HALF, HALF + Q))

        def c_send_rs(si, s):
            if si < 2:
                return lax.rem(my + 2 * N_DEV - 1 - s, N_DEV)
            return lax.rem(my + 1 + s, N_DEV)

        def c_recv_ag(si, t):
            if si < 2:
                return lax.rem(my + 2 * N_DEV - 1 - t, N_DEV)
            return lax.rem(my + 1 + t, N_DEV)

        def start_rdma(si, h, src):
            nbr, _, rb, ss, rs, _ = streams[si]
            pltpu.make_async_remote_copy(
                src_ref=src,
                dst_ref=rb.at[h],
                send_sem=ss.at[h],
                recv_sem=rs.at[h],
                device_id=(nbr,),
                device_id_type=pl.DeviceIdType.MESH,
            ).start()

        def wait_recv(si, h):
            nbr, sb, rb, ss, rs, _ = streams[si]
            pltpu.make_async_remote_copy(
                src_ref=sb.at[0], dst_ref=rb.at[h],
                send_sem=ss.at[h], recv_sem=rs.at[h],
                device_id=(nbr,), device_id_type=pl.DeviceIdType.MESH,
            ).wait_recv()

        def wait_send(si, h, src):
            nbr, _, rb, ss, rs, _ = streams[si]
            pltpu.make_async_remote_copy(
                src_ref=src, dst_ref=rb.at[h],
                send_sem=ss.at[h], recv_sem=rs.at[h],
                device_id=(nbr,), device_id_type=pl.DeviceIdType.MESH,
            ).wait_send()

        ewb = ew_ref[:, :, :].astype(jnp.bfloat16)
        eids = lax.broadcasted_iota(jnp.int32, (1, N_EXP), 1)

        def compute_chunk(c):
            r0 = c * CHUNK
            xc = x_ref[pl.ds(r0, CHUNK), :]
            scores = jnp.dot(
                xc, rw_ref[:, :], preferred_element_type=jnp.float32
            )
            e0 = idx_ref[pl.ds(r0, CHUNK), 0:1]
            e1 = idx_ref[pl.ds(r0, CHUNK), 1:2]
            s0 = jnp.sum(
                jnp.where(e0 == eids, scores, 0.0), axis=1, keepdims=True
            )
            s1 = jnp.sum(
                jnp.where(e1 == eids, scores, 0.0), axis=1, keepdims=True
            )
            w0 = 1.0 / (1.0 + jnp.exp(s1 - s0))
            w1 = 1.0 - w0
            acc = jnp.zeros((CHUNK, D_OUT), jnp.float32)
            for le in range(EXP_PER_DEV):
                ge = my * EXP_PER_DEV + le
                wc = jnp.where(e0 == ge, w0, 0.0) + jnp.where(e1 == ge, w1, 0.0)
                xw = (xc * wc).astype(jnp.bfloat16)
                acc = acc + jnp.dot(
                    xw, ewb[le], preferred_element_type=jnp.float32
                )
            out_ref[pl.ds(r0, CHUNK), :] = acc

        def rows(c):
            return pl.ds(c * CHUNK, CHUNK)

        compute_chunk(lax.rem(my + N_DEV - 1, N_DEV))
        for si in (0, 1):
            c = c_send_rs(si, 0)
            sb = streams[si][1]
            sb[0] = out_ref[rows(c), cols[si]].astype(jnp.bfloat16)
            start_rdma(si, 0, sb.at[0])
        compute_chunk(lax.rem(my + 1, N_DEV))
        for si in (2, 3):
            c = c_send_rs(si, 0)
            sb = streams[si][1]
            sb[0] = out_ref[rows(c), cols[si]].astype(jnp.bfloat16)
            start_rdma(si, 0, sb.at[0])

        compute_chunk(lax.rem(my + N_DEV - 2, N_DEV))
        compute_chunk(my)

        for h in range(N_DEV - 2):
            for si in range(4):
                wait_recv(si, h)
                _, sb, rb, _, _, _ = streams[si]
                c = c_send_rs(si, h + 1)
                tmp = out_ref[rows(c), cols[si]] + rb[h].astype(jnp.float32)
                sb[h + 1] = tmp.astype(jnp.bfloat16)
                start_rdma(si, h + 1, sb.at[h + 1])

        for si in range(4):
            wait_recv(si, N_DEV - 2)
            _, sb, rb, _, _, _ = streams[si]
            tmp = out_ref[rows(my), cols[si]] + rb[N_DEV - 2].astype(
                jnp.float32
            )
            sb[N_DEV - 1] = tmp.astype(jnp.bfloat16)
            start_rdma(si, N_DEV - 1, sb.at[N_DEV - 1])
            out_ref[rows(my), cols[si]] = tmp

        for t in range(1, N_DEV - 1):
            h = N_DEV - 1 + t
            for si in range(4):
                wait_recv(si, h - 1)
                _, _, rb, _, _, _ = streams[si]
                start_rdma(si, h, rb.at[h - 1])
                c = c_recv_ag(si, t - 1)
                out_ref[rows(c), cols[si]] = rb[h - 1].astype(jnp.float32)

        for si in range(4):
            wait_recv(si, N_HOPS - 1)
            _, _, rb, _, _, _ = streams[si]
            c = c_recv_ag(si, N_DEV - 2)
            out_ref[rows(c), cols[si]] = rb[N_HOPS - 1].astype(jnp.float32)

        for si in range(4):
            _, sb, rb, _, _, _ = streams[si]
            for h in range(N_STAGED):
                wait_send(si, h, sb.at[h])
            for h in range(N_STAGED, N_HOPS):
                wait_send(si, h, rb.at[h - 1])

    return pl.pallas_call(
        body,
        out_shape=jax.ShapeDtypeStruct((N_TOK, D_OUT), jnp.float32),
        in_specs=[pl.BlockSpec(memory_space=pltpu.VMEM)] * 4,
        out_specs=pl.BlockSpec(memory_space=pltpu.VMEM),
        scratch_shapes=(
            [pltpu.VMEM((N_STAGED, CHUNK, Q), jnp.bfloat16)] * 4
            + [pltpu.VMEM((N_HOPS, CHUNK, Q), jnp.bfloat16)] * 4
            + [pltpu.SemaphoreType.DMA((N_HOPS,))] * 8
        ),
        compiler_params=pltpu.CompilerParams(collective_id=0),
    )(x, router_W, route_idx, expert_W)


# device time: 59936 ns/iter; 1.0016x vs baseline; 1.0016x over previous
import jax
import jax.numpy as jnp
from jax import lax
from jax.experimental import pallas as pl
from jax.experimental.pallas import tpu as pltpu

N_DEV = 4
N_TOK = 2048
D_IN = 512
D_OUT = 1024
HALF = D_OUT // 2
Q = HALF // 2
N_EXP = 16
EXP_PER_DEV = N_EXP // N_DEV
CHUNK = N_TOK // N_DEV
N_HOPS = 2 * (N_DEV - 1)
N_STAGED = N_DEV


def kernel(x, router_W, route_idx, expert_W):
    def body(x_ref, rw_ref, idx_ref, ew_ref, out_ref, pb_ref,
             sb0, sb1, sb2, sb3, rb0, rb1, rb2, rb3,
             ss0, ss1, ss2, ss3, rs0, rs1, rs2, rs3):
        my = lax.axis_index("i")
        left = lax.rem(my + N_DEV - 1, N_DEV)
        right = lax.rem(my + 1, N_DEV)

        barrier_sem = pltpu.get_barrier_semaphore()
        for nbr in (left, right):
            pl.semaphore_signal(
                barrier_sem, inc=1,
                device_id=(nbr,), device_id_type=pl.DeviceIdType.MESH,
            )
        pl.semaphore_wait(barrier_sem, 2)

        streams = (
            (right, sb0, rb0, ss0, rs0, 0),
            (right, sb1, rb1, ss1, rs1, Q),
            (left, sb2, rb2, ss2, rs2, HALF),
            (left, sb3, rb3, ss3, rs3, HALF + Q),
        )
        cols = tuple(slice(off, off + Q) for off in (0, Q, HALF, HALF + Q))

        def c_send_rs(si, s):
            if si < 2:
                return lax.rem(my + 2 * N_DEV - 1 - s, N_DEV)
            return lax.rem(my + 1 + s, N_DEV)

        def c_recv_ag(si, t):
            if si < 2:
                return lax.rem(my + 2 * N_DEV - 1 - t, N_DEV)
            return lax.rem(my + 1 + t, N_DEV)

        def start_rdma(si, h, src):
            nbr, _, rb, ss, rs, _ = streams[si]
            pltpu.make_async_remote_copy(
                src_ref=src,
                dst_ref=rb.at[h],
                send_sem=ss.at[h],
                recv_sem=rs.at[h],
                device_id=(nbr,),
                device_id_type=pl.DeviceIdType.MESH,
            ).start()

        def wait_recv(si, h):
            nbr, sb, rb, ss, rs, _ = streams[si]
            pltpu.make_async_remote_copy(
                src_ref=sb.at[0], dst_ref=rb.at[h],
                send_sem=ss.at[h], recv_sem=rs.at[h],
                device_id=(nbr,), device_id_type=pl.DeviceIdType.MESH,
            ).wait_recv()

        def wait_send(si, h, src):
            nbr, _, rb, ss, rs, _ = streams[si]
            pltpu.make_async_remote_copy(
                src_ref=src, dst_ref=rb.at[h],
                send_sem=ss.at[h], recv_sem=rs.at[h],
                device_id=(nbr,), device_id_type=pl.DeviceIdType.MESH,
            ).wait_send()

        ewb = ew_ref[:, :, :].astype(jnp.bfloat16)
        eids = lax.broadcasted_iota(jnp.int32, (1, N_EXP), 1)

        def compute_chunk(c):
            r0 = c * CHUNK
            xc = x_ref[pl.ds(r0, CHUNK), :]
            scores = jnp.dot(
                xc, rw_ref[:, :], preferred_element_type=jnp.float32
            )
            e0 = idx_ref[pl.ds(r0, CHUNK), 0:1]
            e1 = idx_ref[pl.ds(r0, CHUNK), 1:2]
            s0 = jnp.sum(
                jnp.where(e0 == eids, scores, 0.0), axis=1, keepdims=True
            )
            s1 = jnp.sum(
                jnp.where(e1 == eids, scores, 0.0), axis=1, keepdims=True
            )
            w0 = 1.0 / (1.0 + jnp.exp(s1 - s0))
            w1 = 1.0 - w0
            wles = []
            for le in range(EXP_PER_DEV):
                ge = my * EXP_PER_DEV + le
                wles.append(
                    jnp.where(e0 == ge, w0, 0.0) + jnp.where(e1 == ge, w1, 0.0)
                )
            w4 = jnp.stack(wles)
            xw4 = (w4 * xc[None, :, :]).astype(jnp.bfloat16)
            acc = jnp.zeros((CHUNK, D_OUT), jnp.float32)
            for le in range(EXP_PER_DEV):
                acc = acc + jnp.dot(
                    xw4[le], ewb[le], preferred_element_type=jnp.float32
                )
            pb_ref[pl.ds(r0, CHUNK), :] = acc.astype(jnp.bfloat16)

        def rows(c):
            return pl.ds(c * CHUNK, CHUNK)

        compute_chunk(lax.rem(my + N_DEV - 1, N_DEV))
        for si in (0, 1):
            c = c_send_rs(si, 0)
            sb = streams[si][1]
            sb[0] = pb_ref[rows(c), cols[si]]
            start_rdma(si, 0, sb.at[0])
        compute_chunk(lax.rem(my + 1, N_DEV))
        for si in (2, 3):
            c = c_send_rs(si, 0)
            sb = streams[si][1]
            sb[0] = pb_ref[rows(c), cols[si]]
            start_rdma(si, 0, sb.at[0])

        compute_chunk(lax.rem(my + N_DEV - 2, N_DEV))
        compute_chunk(my)

        for h in range(N_DEV - 2):
            for si in range(4):
                wait_recv(si, h)
                _, sb, rb, _, _, _ = streams[si]
                c = c_send_rs(si, h + 1)
                tmp = pb_ref[rows(c), cols[si]].astype(jnp.float32) + rb[
                    h
                ].astype(jnp.float32)
                sb[h + 1] = tmp.astype(jnp.bfloat16)
                start_rdma(si, h + 1, sb.at[h + 1])

        for si in range(4):
            wait_recv(si, N_DEV - 2)
            _, sb, rb, _, _, _ = streams[si]
            tmp = pb_ref[rows(my), cols[si]].astype(jnp.float32) + rb[
                N_DEV - 2
            ].astype(jnp.float32)
            sb[N_DEV - 1] = tmp.astype(jnp.bfloat16)
            start_rdma(si, N_DEV - 1, sb.at[N_DEV - 1])
            out_ref[rows(my), cols[si]] = tmp

        for t in range(1, N_DEV - 1):
            h = N_DEV - 1 + t
            for si in range(4):
                wait_recv(si, h - 1)
                _, _, rb, _, _, _ = streams[si]
                start_rdma(si, h, rb.at[h - 1])
                c = c_recv_ag(si, t - 1)
                out_ref[rows(c), cols[si]] = rb[h - 1].astype(jnp.float32)

        for si in range(4):
            wait_recv(si, N_HOPS - 1)
            _, _, rb, _, _, _ = streams[si]
            c = c_recv_ag(si, N_DEV - 2)
            out_ref[rows(c), cols[si]] = rb[N_HOPS - 1].astype(jnp.float32)

        for si in range(4):
            _, sb, rb, _, _, _ = streams[si]
            for h in range(N_STAGED):
                wait_send(si, h, sb.at[h])
            for h in range(N_STAGED, N_HOPS):
                wait_send(si, h, rb.at[h - 1])

    return pl.pallas_call(
        body,
        out_shape=jax.ShapeDtypeStruct((N_TOK, D_OUT), jnp.float32),
        in_specs=[pl.BlockSpec(memory_space=pltpu.VMEM)] * 4,
        out_specs=pl.BlockSpec(memory_space=pltpu.VMEM),
        scratch_shapes=(
            [pltpu.VMEM((N_TOK, D_OUT), jnp.bfloat16)]
            + [pltpu.VMEM((N_STAGED, CHUNK, Q), jnp.bfloat16)] * 4
            + [pltpu.VMEM((N_HOPS, CHUNK, Q), jnp.bfloat16)] * 4
            + [pltpu.SemaphoreType.DMA((N_HOPS,))] * 8
        ),
        compiler_params=pltpu.CompilerParams(collective_id=0),
    )(x, router_W, route_idx, expert_W)


# device time: 58673 ns/iter; 1.0232x vs baseline; 1.0215x over previous
import jax
import jax.numpy as jnp
from jax import lax
from jax.experimental import pallas as pl
from jax.experimental.pallas import tpu as pltpu

N_DEV = 4
N_TOK = 2048
D_IN = 512
D_OUT = 1024
HALF = D_OUT // 2
Q = HALF // 2
N_EXP = 16
EXP_PER_DEV = N_EXP // N_DEV
CHUNK = N_TOK // N_DEV
N_HOPS = 2 * (N_DEV - 1)
N_STAGED = N_DEV


def kernel(x, router_W, route_idx, expert_W):
    def body(x_ref, rw_ref, idx_ref, ew_ref, out_ref,
             sb0, sb1, sb2, sb3, rb0, rb1, rb2, rb3,
             ss0, ss1, ss2, ss3, rs0, rs1, rs2, rs3):
        my = lax.axis_index("i")
        left = lax.rem(my + N_DEV - 1, N_DEV)
        right = lax.rem(my + 1, N_DEV)

        barrier_sem = pltpu.get_barrier_semaphore()
        for nbr in (left, right):
            pl.semaphore_signal(
                barrier_sem, inc=1,
                device_id=(nbr,), device_id_type=pl.DeviceIdType.MESH,
            )
        pl.semaphore_wait(barrier_sem, 2)

        streams = (
            (right, sb0, rb0, ss0, rs0, 0),
            (right, sb1, rb1, ss1, rs1, Q),
            (left, sb2, rb2, ss2, rs2, HALF),
            (left, sb3, rb3, ss3, rs3, HALF + Q),
        )
        cols = tuple(slice(off, off + Q) for off in (0, Q, HALF, HALF + Q))

        def c_send_rs(si, s):
            if si < 2:
                return lax.rem(my + 2 * N_DEV - 1 - s, N_DEV)
            return lax.rem(my + 1 + s, N_DEV)

        def c_recv_ag(si, t):
            if si < 2:
                return lax.rem(my + 2 * N_DEV - 1 - t, N_DEV)
            return lax.rem(my + 1 + t, N_DEV)

        def start_rdma(si, h, src):
            nbr, _, rb, ss, rs, _ = streams[si]
            pltpu.make_async_remote_copy(
                src_ref=src,
                dst_ref=rb.at[h],
                send_sem=ss.at[h],
                recv_sem=rs.at[h],
                device_id=(nbr,),
                device_id_type=pl.DeviceIdType.MESH,
            ).start()

        def wait_recv(si, h):
            nbr, sb, rb, ss, rs, _ = streams[si]
            pltpu.make_async_remote_copy(
                src_ref=sb.at[0], dst_ref=rb.at[h],
                send_sem=ss.at[h], recv_sem=rs.at[h],
                device_id=(nbr,), device_id_type=pl.DeviceIdType.MESH,
            ).wait_recv()

        def wait_send(si, h, src):
            nbr, _, rb, ss, rs, _ = streams[si]
            pltpu.make_async_remote_copy(
                src_ref=src, dst_ref=rb.at[h],
                send_sem=ss.at[h], recv_sem=rs.at[h],
                device_id=(nbr,), device_id_type=pl.DeviceIdType.MESH,
            ).wait_send()

        ewb = ew_ref[:, :, :].astype(jnp.bfloat16)
        eids = lax.broadcasted_iota(jnp.int32, (1, N_EXP), 1)

        def compute_chunk(c):
            r0 = c * CHUNK
            xc = x_ref[pl.ds(r0, CHUNK), :]
            scores = jnp.dot(
                xc, rw_ref[:, :], preferred_element_type=jnp.float32
            )
            e0 = idx_ref[pl.ds(r0, CHUNK), 0:1]
            e1 = idx_ref[pl.ds(r0, CHUNK), 1:2]
            s0 = jnp.sum(
                jnp.where(e0 == eids, scores, 0.0), axis=1, keepdims=True
            )
            s1 = jnp.sum(
                jnp.where(e1 == eids, scores, 0.0), axis=1, keepdims=True
            )
            w0 = 1.0 / (1.0 + jnp.exp(s1 - s0))
            w1 = 1.0 - w0
            wles = []
            for le in range(EXP_PER_DEV):
                ge = my * EXP_PER_DEV + le
                wles.append(
                    jnp.where(e0 == ge, w0, 0.0) + jnp.where(e1 == ge, w1, 0.0)
                )
            w4 = jnp.stack(wles)
            xw4 = (w4 * xc[None, :, :]).astype(jnp.bfloat16)
            acc = jnp.zeros((CHUNK, D_OUT), jnp.float32)
            for le in range(EXP_PER_DEV):
                acc = acc + jnp.dot(
                    xw4[le], ewb[le], preferred_element_type=jnp.float32
                )
            out_ref[pl.ds(r0, CHUNK), :] = acc.astype(jnp.bfloat16)

        def rows(c):
            return pl.ds(c * CHUNK, CHUNK)

        compute_chunk(lax.rem(my + N_DEV - 1, N_DEV))
        for si in (0, 1):
            c = c_send_rs(si, 0)
            sb = streams[si][1]
            sb[0] = out_ref[rows(c), cols[si]]
            start_rdma(si, 0, sb.at[0])
        compute_chunk(lax.rem(my + 1, N_DEV))
        for si in (2, 3):
            c = c_send_rs(si, 0)
            sb = streams[si][1]
            sb[0] = out_ref[rows(c), cols[si]]
            start_rdma(si, 0, sb.at[0])

        compute_chunk(lax.rem(my + N_DEV - 2, N_DEV))
        compute_chunk(my)

        for h in range(N_DEV - 2):
            for si in range(4):
                wait_recv(si, h)
                _, sb, rb, _, _, _ = streams[si]
                c = c_send_rs(si, h + 1)
                tmp = out_ref[rows(c), cols[si]].astype(jnp.float32) + rb[
                    h
                ].astype(jnp.float32)
                sb[h + 1] = tmp.astype(jnp.bfloat16)
                start_rdma(si, h + 1, sb.at[h + 1])

        for si in range(4):
            wait_recv(si, N_DEV - 2)
            _, sb, rb, _, _, _ = streams[si]
            tmp = out_ref[rows(my), cols[si]].astype(jnp.float32) + rb[
                N_DEV - 2
            ].astype(jnp.float32)
            tmpb = tmp.astype(jnp.bfloat16)
            sb[N_DEV - 1] = tmpb
            start_rdma(si, N_DEV - 1, sb.at[N_DEV - 1])
            out_ref[rows(my), cols[si]] = tmpb

        for t in range(1, N_DEV - 1):
            h = N_DEV - 1 + t
            for si in range(4):
                wait_recv(si, h - 1)
                _, _, rb, _, _, _ = streams[si]
                start_rdma(si, h, rb.at[h - 1])
                c = c_recv_ag(si, t - 1)
                out_ref[rows(c), cols[si]] = rb[h - 1]

        for si in range(4):
            wait_recv(si, N_HOPS - 1)
            _, _, rb, _, _, _ = streams[si]
            c = c_recv_ag(si, N_DEV - 2)
            out_ref[rows(c), cols[si]] = rb[N_HOPS - 1]

        for si in range(4):
            _, sb, rb, _, _, _ = streams[si]
            for h in range(N_STAGED):
                wait_send(si, h, sb.at[h])
            for h in range(N_STAGED, N_HOPS):
                wait_send(si, h, rb.at[h - 1])

    return pl.pallas_call(
        body,
        out_shape=jax.ShapeDtypeStruct((N_TOK, D_OUT), jnp.bfloat16),
        in_specs=[pl.BlockSpec(memory_space=pltpu.VMEM)] * 4,
        out_specs=pl.BlockSpec(memory_space=pltpu.VMEM),
        scratch_shapes=(
            [pltpu.VMEM((N_STAGED, CHUNK, Q), jnp.bfloat16)] * 4
            + [pltpu.VMEM((N_HOPS, CHUNK, Q), jnp.bfloat16)] * 4
            + [pltpu.SemaphoreType.DMA((N_HOPS,))] * 8
        ),
        compiler_params=pltpu.CompilerParams(collective_id=0),
    )(x, router_W, route_idx, expert_W)
